# Initial kernel scaffold; baseline (speedup 1.0000x reference)
import jax
import jax.numpy as jnp
from jax.experimental import pallas as pl


def kernel(x, edge_index, edge_weight, W, a_l, a_r, bias):
    raise NotImplementedError("write your pallas kernel here")


# speedup vs baseline: 42.5102x; 42.5102x over previous
import functools

import jax
import jax.numpy as jnp
from jax import lax
from jax.experimental import pallas as pl
from jax.experimental.pallas import tpu as pltpu
from jax.experimental.pallas import tpu_sc as plsc

N = 10000
E = 320000
H = 4
F = 32
HF = H * F
NEG = 0.2
SHIFT = 9.0

NW = 32
EPW = 10240
E_PAD = NW * EPW
CH = 128
NCHUNK = EPW // CH
NBLK = 79


def _tc_prep(x_ref, wf_ref, alr_ref, wh_ref, e8_ref, c_ref):
    x = x_ref[...]
    wh = jnp.dot(x, wf_ref[...], preferred_element_type=jnp.float32)
    wh_ref[...] = wh
    e8 = jnp.dot(wh, alr_ref[...], preferred_element_type=jnp.float32)
    e8_ref[...] = e8
    c = jnp.max(e8[:, 0:4], axis=0) - SHIFT
    cpad = jnp.concatenate([c, jnp.zeros((4,), jnp.float32)])
    c_ref[...] = jnp.broadcast_to(cpad[:, None], (8, 128))


def _tc_finish(op_ref, sp_ref, b_ref, o_ref):
    acc = op_ref[0] + op_ref[1]
    s4 = sp_ref[0, :, 0:4] + sp_ref[1, :, 0:4] + 1e-8
    d = 1.0 / s4
    dfull = jnp.reshape(jnp.broadcast_to(d[:, :, None], (N, 4, 32)), (N, HF))
    o_ref[...] = acc * dfull + b_ref[...]


def _sc_edges(wh_hbm, e8_hbm, c_hbm, row_hbm, col_hbm, ew_hbm,
              outp_hbm, sp_hbm,
              cbuf, rowbuf, colbuf, ewbuf, sstage, whbuf,
              elbuf, erbuf, out_sh, s_sh, sem, sem_e, sem_w):
    cid = lax.axis_index("c")
    sid = lax.axis_index("s")
    wid = cid * 16 + sid
    zeros = jnp.zeros((16,), jnp.float32)

    def _zrow(i, _):
        for j in range(8):
            whbuf[i, pl.ds(j * 16, 16)] = zeros
        sstage[i, pl.ds(0, 16)] = zeros
        return 0

    lax.fori_loop(0, CH, _zrow, 0)

    for b in range(NBLK):
        @pl.when(sid == (b % 16))
        def _zero_blk(b=b):
            base = b * 128
            sz = 128 if b < NBLK - 1 else N - 128 * (NBLK - 1)
            pltpu.sync_copy(whbuf.at[pl.ds(0, sz), :],
                            out_sh.at[pl.ds(base, sz), :])
            pltpu.sync_copy(sstage.at[pl.ds(0, sz), :],
                            s_sh.at[pl.ds(base, sz), :])

    pltpu.sync_copy(c_hbm, cbuf)

    plsc.subcore_barrier()

    iota16 = lax.iota(jnp.int32, 16)

    def _chunk(ci, _):
        e0 = wid * EPW + ci * CH
        cpr = pltpu.async_copy(row_hbm.at[pl.ds(e0, CH)], rowbuf, sem)
        cpc = pltpu.async_copy(col_hbm.at[pl.ds(e0, CH)], colbuf, sem)
        cpe = pltpu.async_copy(ew_hbm.at[pl.ds(e0, CH)], ewbuf, sem)
        cpr.wait()
        cpc.wait()
        cpe.wait()
        cpw = pltpu.async_copy(wh_hbm.at[rowbuf], whbuf, sem_w)
        cpl = pltpu.async_copy(e8_hbm.at[rowbuf], elbuf, sem_e)
        cpd = pltpu.async_copy(e8_hbm.at[colbuf], erbuf, sem_e)
        cpl.wait()
        cpd.wait()

        for k in range(8):
            off = k * 16
            idx = off + iota16
            w16 = ewbuf[pl.ds(off, 16)]
            wc = jnp.maximum(w16, 1e-8)
            for h in range(4):
                a = plsc.load_gather(elbuf, [idx, jnp.full((16,), h, jnp.int32)])
                b = plsc.load_gather(erbuf, [idx, jnp.full((16,), 4 + h, jnp.int32)])
                ch = cbuf[h, pl.ds(0, 16)]
                s1 = a + b
                l1 = jnp.where(s1 >= 0, s1, NEG * s1)
                s2 = ch + b
                l2 = jnp.where(s2 >= 0, s2, NEG * s2)
                p = wc * jnp.exp(l1 - l2)
                p = jnp.where(w16 < 0, 0.0, p)
                plsc.store_scatter(
                    sstage, [idx, jnp.full((16,), h, jnp.int32)], p)

        pltpu.sync_copy(sstage, s_sh.at[colbuf], add=True)

        cpw.wait()

        def _scale(j, _):
            jv = jnp.full((16,), j, jnp.int32)
            for h in range(4):
                pv = plsc.load_gather(sstage, [jv, jnp.full((16,), h, jnp.int32)])
                for s_ in range(2):
                    sl = pl.ds(h * 32 + s_ * 16, 16)
                    whbuf[j, sl] = whbuf[j, sl] * pv
            return 0

        lax.fori_loop(0, CH, _scale, 0)

        pltpu.sync_copy(whbuf, out_sh.at[colbuf], add=True)
        return 0

    lax.fori_loop(0, NCHUNK, _chunk, 0)

    plsc.subcore_barrier()

    for b in range(NBLK):
        @pl.when(sid == (b % 16))
        def _write_blk(b=b):
            base = b * 128
            sz = 128 if b < NBLK - 1 else N - 128 * (NBLK - 1)
            pltpu.sync_copy(out_sh.at[pl.ds(base, sz), :],
                            outp_hbm.at[cid, pl.ds(base, sz), :])
            pltpu.sync_copy(s_sh.at[pl.ds(base, sz), :],
                            sp_hbm.at[cid, pl.ds(base, sz), :])


_sc_call = pl.kernel(
    _sc_edges,
    out_type=(
        jax.ShapeDtypeStruct((2, N, HF), jnp.float32),
        jax.ShapeDtypeStruct((2, N, 16), jnp.float32),
    ),
    mesh=plsc.VectorSubcoreMesh(core_axis_name="c", subcore_axis_name="s"),
    compiler_params=pltpu.CompilerParams(needs_layout_passes=False,
                                         use_tc_tiling_on_sc=False),
    scratch_types=[
        pltpu.VMEM((8, 128), jnp.float32),
        pltpu.VMEM((CH,), jnp.int32),
        pltpu.VMEM((CH,), jnp.int32),
        pltpu.VMEM((CH,), jnp.float32),
        pltpu.VMEM((CH, 16), jnp.float32),
        pltpu.VMEM((CH, HF), jnp.float32),
        pltpu.VMEM((CH, 8), jnp.float32),
        pltpu.VMEM((CH, 8), jnp.float32),
        pltpu.VMEM_SHARED((N, HF), jnp.float32),
        pltpu.VMEM_SHARED((N, 16), jnp.float32),
        pltpu.SemaphoreType.DMA,
        pltpu.SemaphoreType.DMA,
        pltpu.SemaphoreType.DMA,
    ],
)


def kernel(x, edge_index, edge_weight, W, a_l, a_r, bias):
    wf = jnp.transpose(W, (1, 0, 2)).reshape(HF, HF)
    al2 = a_l[:, :, 0]
    ar2 = a_r[:, :, 0]
    alr = jnp.zeros((HF, 8), jnp.float32)
    for h in range(H):
        alr = alr.at[h * F:(h + 1) * F, h].set(al2[h])
        alr = alr.at[h * F:(h + 1) * F, 4 + h].set(ar2[h])

    wh, e8, c = pl.pallas_call(
        _tc_prep,
        out_shape=(
            jax.ShapeDtypeStruct((N, HF), jnp.float32),
            jax.ShapeDtypeStruct((N, 8), jnp.float32),
            jax.ShapeDtypeStruct((8, 128), jnp.float32),
        ),
    )(x, wf, alr)

    row = jnp.concatenate([edge_index[0], jnp.zeros((E_PAD - E,), jnp.int32)])
    col = jnp.concatenate([edge_index[1], jnp.zeros((E_PAD - E,), jnp.int32)])
    ew = jnp.concatenate([edge_weight, jnp.full((E_PAD - E,), -1.0, jnp.float32)])

    outp, sp = _sc_call(wh, e8, c, row, col, ew)

    out = pl.pallas_call(
        _tc_finish,
        out_shape=jax.ShapeDtypeStruct((N, HF), jnp.float32),
    )(outp, sp, bias.reshape(1, HF))
    return out


# speedup vs baseline: 66.0897x; 1.5547x over previous
import jax
import jax.numpy as jnp
from jax import lax
from jax.experimental import pallas as pl
from jax.experimental.pallas import tpu as pltpu
from jax.experimental.pallas import tpu_sc as plsc

N = 10000
E = 320000
H = 4
F = 32
HF = H * F
NEG = 0.2
SHIFT = 9.0

NW = 32
EPW = 10240
E_PAD = NW * EPW
CH = 128
NCHUNK = EPW // CH
NBLK = 79
PACK = 16384


def _tc_prep(x_ref, wf_ref, alr_ref, wh_ref, e8_ref, c_ref):
    x = x_ref[...]
    wh = jnp.dot(x, wf_ref[...], preferred_element_type=jnp.float32)
    wh_ref[...] = wh
    e8 = jnp.dot(wh, alr_ref[...], preferred_element_type=jnp.float32)
    e8_ref[...] = e8
    c = jnp.max(e8[:, 0:4], axis=0) - SHIFT
    cpad = jnp.concatenate([c, jnp.zeros((4,), jnp.float32)])
    c_ref[...] = jnp.broadcast_to(cpad[:, None], (8, 128))


def _tc_finish(op_ref, sp_ref, b_ref, o_ref):
    acc = op_ref[0] + op_ref[1]
    s4 = sp_ref[0, :, 0:4] + sp_ref[1, :, 0:4] + 1e-8
    d = 1.0 / s4
    dfull = jnp.reshape(jnp.broadcast_to(d[:, :, None], (N, 4, 32)), (N, HF))
    o_ref[...] = acc * dfull + b_ref[...]


def _sc_edges(wh_hbm, e8_hbm, c_hbm, rc_hbm, ew_hbm,
              outp_hbm, sp_hbm,
              cbuf, sstage,
              rcb0, rowb0, colb0, ewb0, whbuf0, elbuf0, erbuf0,
              rcb1, rowb1, colb1, ewb1, whbuf1, elbuf1, erbuf1,
              out_sh, s_sh,
              semi0, semi1, seme0, seme1, semw0, semw1):
    cid = lax.axis_index("c")
    sid = lax.axis_index("s")
    wid = cid * 16 + sid
    zeros = jnp.zeros((16,), jnp.float32)

    def _zrow(i, _):
        for j in range(8):
            whbuf0[i, pl.ds(j * 16, 16)] = zeros
        sstage[i, pl.ds(0, 16)] = zeros
        return 0

    lax.fori_loop(0, CH, _zrow, 0)

    for b in range(NBLK):
        @pl.when(sid == (b % 16))
        def _zero_blk(b=b):
            base = b * 128
            sz = 128 if b < NBLK - 1 else N - 128 * (NBLK - 1)
            pltpu.sync_copy(whbuf0.at[pl.ds(0, sz), :],
                            out_sh.at[pl.ds(base, sz), :])
            pltpu.sync_copy(sstage.at[pl.ds(0, sz), :],
                            s_sh.at[pl.ds(base, sz), :])

    pltpu.sync_copy(c_hbm, cbuf)

    plsc.subcore_barrier()

    iota16 = lax.iota(jnp.int32, 16)

    def _issue(ci, rcb, rowb, colb, ewb, whb, elb, erb, semi, semw, seme):
        pltpu.make_async_copy(rc_hbm.at[wid, ci], rcb, semi).wait()

        def _unpack(k, _):
            off = k * 16
            v = rcb[pl.ds(off, 16)]
            rowb[pl.ds(off, 16)] = lax.shift_right_logical(v, 14)
            colb[pl.ds(off, 16)] = lax.bitwise_and(v, PACK - 1)
            return 0

        lax.fori_loop(0, 8, _unpack, 0)
        pltpu.async_copy(ew_hbm.at[wid, ci], ewb, seme)
        pltpu.async_copy(wh_hbm.at[rowb], whb, semw)
        pltpu.async_copy(e8_hbm.at[rowb], elb, seme)
        pltpu.async_copy(e8_hbm.at[colb], erb, seme)

        @pl.when(ci + 2 < NCHUNK)
        def _():
            pltpu.async_copy(rc_hbm.at[wid, ci + 2], rcb, semi)

    def _compute(ci, rowb, colb, ewb, whb, elb, erb, semw, seme):
        pltpu.make_async_copy(ew_hbm.at[wid, ci], ewb, seme).wait()
        pltpu.make_async_copy(e8_hbm.at[rowb], elb, seme).wait()
        pltpu.make_async_copy(e8_hbm.at[colb], erb, seme).wait()
        def _pgroup(k, _):
            off = k * 16
            idx = off + iota16
            w16 = ewb[pl.ds(off, 16)]
            wc = jnp.maximum(w16, 1e-8)
            for h in range(4):
                a = plsc.load_gather(elb, [idx, jnp.full((16,), h, jnp.int32)])
                b = plsc.load_gather(erb, [idx, jnp.full((16,), 4 + h, jnp.int32)])
                ch = cbuf[h, pl.ds(0, 16)]
                s1 = a + b
                l1 = jnp.where(s1 >= 0, s1, NEG * s1)
                s2 = ch + b
                l2 = jnp.where(s2 >= 0, s2, NEG * s2)
                p = wc * jnp.exp(l1 - l2)
                p = jnp.where(w16 < 0, 0.0, p)
                plsc.store_scatter(
                    sstage, [idx, jnp.full((16,), h, jnp.int32)], p)
            return 0

        lax.fori_loop(0, 8, _pgroup, 0)

        pltpu.sync_copy(sstage, s_sh.at[colb], add=True)

        pltpu.make_async_copy(wh_hbm.at[rowb], whb, semw).wait()

        def _scale(j, _):
            jv = jnp.full((16,), j, jnp.int32)
            for h in range(4):
                pv = plsc.load_gather(
                    sstage, [jv, jnp.full((16,), h, jnp.int32)])
                for s_ in range(2):
                    sl = pl.ds(h * 32 + s_ * 16, 16)
                    whb[j, sl] = whb[j, sl] * pv
            return 0

        lax.fori_loop(0, CH, _scale, 0)

        pltpu.sync_copy(whb, out_sh.at[colb], add=True)

    pltpu.async_copy(rc_hbm.at[wid, 0], rcb0, semi0)
    pltpu.async_copy(rc_hbm.at[wid, 1], rcb1, semi1)
    _issue(0, rcb0, rowb0, colb0, ewb0, whbuf0, elbuf0, erbuf0,
           semi0, semw0, seme0)

    def _pair(i, _):
        ci0 = 2 * i
        _issue(ci0 + 1, rcb1, rowb1, colb1, ewb1, whbuf1, elbuf1, erbuf1,
               semi1, semw1, seme1)
        _compute(ci0, rowb0, colb0, ewb0, whbuf0, elbuf0, erbuf0,
                 semw0, seme0)

        @pl.when(i < NCHUNK // 2 - 1)
        def _():
            _issue(ci0 + 2, rcb0, rowb0, colb0, ewb0, whbuf0, elbuf0, erbuf0,
                   semi0, semw0, seme0)

        _compute(ci0 + 1, rowb1, colb1, ewb1, whbuf1, elbuf1, erbuf1,
                 semw1, seme1)
        return 0

    lax.fori_loop(0, NCHUNK // 2, _pair, 0)

    plsc.subcore_barrier()

    for b in range(NBLK):
        @pl.when(sid == (b % 16))
        def _write_blk(b=b):
            base = b * 128
            sz = 128 if b < NBLK - 1 else N - 128 * (NBLK - 1)
            pltpu.sync_copy(out_sh.at[pl.ds(base, sz), :],
                            outp_hbm.at[cid, pl.ds(base, sz), :])
            pltpu.sync_copy(s_sh.at[pl.ds(base, sz), :],
                            sp_hbm.at[cid, pl.ds(base, sz), :])


_sc_call = pl.kernel(
    _sc_edges,
    out_type=(
        jax.ShapeDtypeStruct((2, N, HF), jnp.float32),
        jax.ShapeDtypeStruct((2, N, 16), jnp.float32),
    ),
    mesh=plsc.VectorSubcoreMesh(core_axis_name="c", subcore_axis_name="s"),
    compiler_params=pltpu.CompilerParams(needs_layout_passes=False,
                                         use_tc_tiling_on_sc=False),
    scratch_types=[
        pltpu.VMEM((8, 128), jnp.float32),
        pltpu.VMEM((CH, 16), jnp.float32),
        pltpu.VMEM((CH,), jnp.int32),
        pltpu.VMEM((CH,), jnp.int32),
        pltpu.VMEM((CH,), jnp.int32),
        pltpu.VMEM((CH,), jnp.float32),
        pltpu.VMEM((CH, HF), jnp.float32),
        pltpu.VMEM((CH, 8), jnp.float32),
        pltpu.VMEM((CH, 8), jnp.float32),
        pltpu.VMEM((CH,), jnp.int32),
        pltpu.VMEM((CH,), jnp.int32),
        pltpu.VMEM((CH,), jnp.int32),
        pltpu.VMEM((CH,), jnp.float32),
        pltpu.VMEM((CH, HF), jnp.float32),
        pltpu.VMEM((CH, 8), jnp.float32),
        pltpu.VMEM((CH, 8), jnp.float32),
        pltpu.VMEM_SHARED((N, HF), jnp.float32),
        pltpu.VMEM_SHARED((N, 16), jnp.float32),
        pltpu.SemaphoreType.DMA,
        pltpu.SemaphoreType.DMA,
        pltpu.SemaphoreType.DMA,
        pltpu.SemaphoreType.DMA,
        pltpu.SemaphoreType.DMA,
        pltpu.SemaphoreType.DMA,
    ],
)


def kernel(x, edge_index, edge_weight, W, a_l, a_r, bias):
    wf = jnp.transpose(W, (1, 0, 2)).reshape(HF, HF)
    al2 = a_l[:, :, 0]
    ar2 = a_r[:, :, 0]
    alr = jnp.zeros((HF, 8), jnp.float32)
    for h in range(H):
        alr = alr.at[h * F:(h + 1) * F, h].set(al2[h])
        alr = alr.at[h * F:(h + 1) * F, 4 + h].set(ar2[h])

    wh, e8, c = pl.pallas_call(
        _tc_prep,
        out_shape=(
            jax.ShapeDtypeStruct((N, HF), jnp.float32),
            jax.ShapeDtypeStruct((N, 8), jnp.float32),
            jax.ShapeDtypeStruct((8, 128), jnp.float32),
        ),
    )(x, wf, alr)

    row = jnp.concatenate([edge_index[0], jnp.zeros((E_PAD - E,), jnp.int32)])
    col = jnp.concatenate([edge_index[1], jnp.zeros((E_PAD - E,), jnp.int32)])
    ew = jnp.concatenate([edge_weight, jnp.full((E_PAD - E,), -1.0, jnp.float32)])
    rc = row * PACK + col

    outp, sp = _sc_call(wh, e8, c,
                        rc.reshape(NW, NCHUNK, CH),
                        ew.reshape(NW, NCHUNK, CH))

    out = pl.pallas_call(
        _tc_finish,
        out_shape=jax.ShapeDtypeStruct((N, HF), jnp.float32),
    )(outp, sp, bias.reshape(1, HF))
    return out
